# baseline (speedup 1.0000x reference)
import functools

import jax
import jax.numpy as jnp
from jax import lax
from jax.experimental import pallas as pl
from jax.experimental.pallas import tpu as pltpu
from jax.experimental.pallas import tpu_sc as plsc

H, W, KS = 2048, 2048, 8
N = 65536
R = H * W
NC, NS, L = 2, 16, 16
CHUNK = 131072
NCHUNK = R // CHUNK
CPS = NCHUNK // NC
STRIPE = CHUNK // NS
PTS = N // NS
ZROWS = 1024

_mesh = plsc.VectorSubcoreMesh(
    core_axis_name="c", subcore_axis_name="s", num_cores=NC, num_subcores=NS
)


@functools.partial(
    pl.kernel,
    out_type=jax.ShapeDtypeStruct((R, KS), jnp.float32),
    mesh=_mesh,
    scratch_types=[
        pltpu.VMEM((PTS,), jnp.int32),
        pltpu.VMEM((PTS,), jnp.int32),
        pltpu.VMEM((PTS,), jnp.int32),
        pltpu.VMEM((PTS + L,), jnp.int32),
        pltpu.VMEM((PTS + L,), jnp.int32),
        pltpu.VMEM((ZROWS, KS), jnp.float32),
        pltpu.VMEM((L, KS), jnp.float32),
        pltpu.VMEM_SHARED((CHUNK + NS, KS), jnp.float32),
    ],
    compiler_params=pltpu.CompilerParams(
        needs_layout_passes=False, use_tc_tiling_on_sc=False
    ),
)
def _scatter_kernel(xcol, ycol, vals, out, xs_v, ys_v, idx_v, ids_v, moff_v,
                    zero_t, vrow, acc):
    c = lax.axis_index("c")
    s = lax.axis_index("s")
    iota = lax.iota(jnp.int32, L)
    zero_i = jnp.zeros((L,), jnp.int32)
    zero_f = jnp.zeros((L,), jnp.float32)

    pltpu.sync_copy(xcol.at[pl.ds(s * PTS, PTS)], xs_v)
    pltpu.sync_copy(ycol.at[pl.ds(s * PTS, PTS)], ys_v)

    def mk_idx(i, _):
        lanes = iota + i * L
        xs = plsc.load_gather(xs_v, [lanes])
        ys = plsc.load_gather(ys_v, [lanes])
        plsc.store_scatter(idx_v, [lanes], xs * W + ys)
        return 0

    lax.fori_loop(0, PTS // L, mk_idx, 0)

    def mk_zero(i, _):
        f = iota + i * L
        plsc.store_scatter(zero_t, [f >> 3, f & 7], zero_f)
        return 0

    lax.fori_loop(0, ZROWS * KS // L, mk_zero, 0)

    def z0(j, _):
        pltpu.sync_copy(zero_t, acc.at[pl.ds(s * STRIPE + j * ZROWS, ZROWS)])
        return 0

    lax.fori_loop(0, STRIPE // ZROWS, z0, 0)

    def chunk_body(lc, prev_off):
        cb = (c * CPS + lc) * CHUNK

        plsc.subcore_barrier()

        def zt(g, _):
            movec = plsc.load_gather(moff_v, [iota + g * L])
            pltpu.sync_copy(zero_t.at[pl.ds(0, L)], acc.at[movec])
            return 0

        lax.fori_loop(0, (prev_off + L - 1) >> 4, zt, 0)

        def scan(i, off):
            lanes = iota + i * L
            iv = plsc.load_gather(idx_v, [lanes])
            rel = iv - cb
            m = (rel >= 0) & (rel < CHUNK)
            mi = jnp.where(m, 1, 0).astype(jnp.int32)
            pos = off + plsc.cumsum(mi) - mi
            plsc.store_scatter(ids_v, [pos], lanes + s * PTS, mask=m)
            plsc.store_scatter(moff_v, [pos], rel, mask=m)
            return off + jnp.sum(mi)

        off = lax.fori_loop(0, PTS // L, scan, jnp.int32(0))

        opos = off + iota
        plsc.store_scatter(moff_v, [opos], zero_i + (CHUNK + s))
        plsc.store_scatter(ids_v, [opos], zero_i + s * PTS)

        plsc.subcore_barrier()

        def accum(g, _):
            lanes = iota + g * L
            idvec = plsc.load_gather(ids_v, [lanes])
            movec = plsc.load_gather(moff_v, [lanes])
            pltpu.sync_copy(vals.at[idvec], vrow)
            pltpu.sync_copy(vrow, acc.at[movec], add=True)
            return 0

        lax.fori_loop(0, (off + L - 1) >> 4, accum, 0)

        plsc.subcore_barrier()

        pltpu.sync_copy(acc.at[pl.ds(s * STRIPE, STRIPE)],
                        out.at[pl.ds(cb + s * STRIPE, STRIPE)])
        return off

    lax.fori_loop(0, CPS, chunk_body, jnp.int32(0))


def kernel(coords, vals):
    xcol = coords[:, 0]
    ycol = coords[:, 1]
    dense = _scatter_kernel(xcol, ycol, vals)
    return dense.reshape(H, W, KS)


# speedup vs baseline: 5.1262x; 5.1262x over previous
import functools

import jax
import jax.numpy as jnp
from jax import lax
from jax.experimental import pallas as pl
from jax.experimental.pallas import tpu as pltpu
from jax.experimental.pallas import tpu_sc as plsc

H, W, KS = 2048, 2048, 8
N = 65536
E = H * W * KS
VE = N * KS
NC, NS, L = 2, 16, 16
CHUNK = 1048576
NCHUNK = E // CHUNK
CPS = NCHUNK // NC
SSZ = CHUNK // NS
PTS = N // NS
ZROWS = 8192
G = 128
LSZ = PTS + G

_mesh = plsc.VectorSubcoreMesh(
    core_axis_name="c", subcore_axis_name="s", num_cores=NC, num_subcores=NS
)


@functools.partial(
    pl.kernel,
    out_type=jax.ShapeDtypeStruct((E,), jnp.float32),
    mesh=_mesh,
    scratch_types=[
        pltpu.VMEM((PTS,), jnp.int32),
        pltpu.VMEM((PTS,), jnp.int32),
        pltpu.VMEM((PTS,), jnp.int32),
        pltpu.VMEM((PTS,), jnp.int32),
        pltpu.VMEM((LSZ,), jnp.int32),
        pltpu.VMEM((LSZ,), jnp.int32),
        pltpu.VMEM((KS, G), jnp.int32),
        pltpu.VMEM((KS, G), jnp.int32),
        pltpu.VMEM((KS, G), jnp.float32),
        pltpu.VMEM((ZROWS,), jnp.float32),
        pltpu.VMEM_SHARED((CHUNK + NS * 1024,), jnp.float32),
        pltpu.SemaphoreType.DMA,
        pltpu.SemaphoreType.DMA,
        pltpu.SemaphoreType.DMA,
        pltpu.SemaphoreType.DMA,
    ],
    compiler_params=pltpu.CompilerParams(needs_layout_passes=False),
)
def _scatter_kernel(xcol, ycol, vals1, out1, xs_v, ys_v, eb_v, vb_v, ebl, vbl,
                    ekidx, vkidx, crows, zbuf, acc, gsem, ssem, zsem, wsem):
    c = lax.axis_index("c")
    s = lax.axis_index("s")
    iota = lax.iota(jnp.int32, L)
    zero_f = jnp.zeros((L,), jnp.float32)

    pltpu.sync_copy(xcol.at[pl.ds(s * PTS, PTS)], xs_v)
    pltpu.sync_copy(ycol.at[pl.ds(s * PTS, PTS)], ys_v)

    def mk_base(i, _):
        lanes = iota + i * L
        xs = plsc.load_gather(xs_v, [lanes])
        ys = plsc.load_gather(ys_v, [lanes])
        plsc.store_scatter(eb_v, [lanes],
                           xs * (W * KS) + ((ys >> 7) << 10) + (ys & 127))
        p = lanes + s * PTS
        plsc.store_scatter(vb_v, [lanes], ((p >> 7) << 10) + (p & 127))
        return 0

    lax.fori_loop(0, PTS // L, mk_base, 0)

    def mk_zero(i, _):
        plsc.store_scatter(zbuf, [iota + i * L], zero_f)
        return 0

    lax.fori_loop(0, ZROWS // L, mk_zero, 0)

    def z0(j, _):
        pltpu.sync_copy(zbuf, acc.at[pl.ds(s * SSZ + j * ZROWS, ZROWS)])
        return 0

    lax.fori_loop(0, SSZ // ZROWS, z0, 0)

    ob0 = c * CPS * CHUNK + s * SSZ
    pltpu.async_copy(acc.at[pl.ds(s * SSZ, SSZ)], out1.at[pl.ds(ob0, SSZ)],
                     wsem)

    def expand_k(dst, j, base16):
        for k in range(KS):
            plsc.store_scatter(dst, [jnp.zeros((L,), jnp.int32) + k,
                                     j * L + iota], base16 + k * G)

    def chunk_body(lc, prev_off):
        cb = (c * CPS + lc) * CHUNK

        pltpu.make_async_copy(acc.at[pl.ds(s * SSZ, SSZ)],
                              out1.at[pl.ds(s * SSZ, SSZ)], wsem).wait()
        plsc.subcore_barrier()

        def zt(g, _):
            for j in range(G // L):
                e16 = plsc.load_gather(ebl, [g * G + j * L + iota])
                expand_k(ekidx, j, e16)
            for k in range(KS):
                pltpu.async_copy(zbuf.at[pl.ds(0, G)],
                                 acc.at[ekidx.at[k]], zsem)
            for k in range(KS):
                pltpu.make_async_copy(zbuf.at[pl.ds(0, G)],
                                      acc.at[ekidx.at[k]], zsem).wait()
            return 0

        lax.fori_loop(0, (prev_off + G - 1) >> 7, zt, 0)

        def scan(i, off):
            lanes = iota + i * L
            eb = plsc.load_gather(eb_v, [lanes])
            rel = eb - cb
            m = (rel >= 0) & (rel < CHUNK)
            mi = jnp.where(m, 1, 0).astype(jnp.int32)
            pos = off + plsc.cumsum(mi) - mi
            plsc.store_scatter(ebl, [pos], rel, mask=m)
            vb = plsc.load_gather(vb_v, [lanes])
            plsc.store_scatter(vbl, [pos], vb, mask=m)
            return off + jnp.sum(mi)

        off = lax.fori_loop(0, PTS // L, scan, jnp.int32(0))

        for j in range(G // L):
            opos = off + j * L + iota
            plsc.store_scatter(ebl, [opos],
                               jnp.zeros((L,), jnp.int32) + (CHUNK + s * 1024))
            plsc.store_scatter(vbl, [opos],
                               jnp.zeros((L,), jnp.int32) + s * (PTS * KS))

        plsc.subcore_barrier()

        def accum(g, _):
            for j in range(G // L):
                lanes = g * G + j * L + iota
                e16 = plsc.load_gather(ebl, [lanes])
                v16 = plsc.load_gather(vbl, [lanes])
                expand_k(ekidx, j, e16)
                expand_k(vkidx, j, v16)
            for k in range(KS):
                pltpu.async_copy(vals1.at[vkidx.at[k]], crows.at[k], gsem)
            for k in range(KS):
                pltpu.make_async_copy(vals1.at[vkidx.at[k]], crows.at[k],
                                      gsem).wait()
            for k in range(KS):
                pltpu.async_copy(crows.at[k], acc.at[ekidx.at[k]], ssem,
                                 add=True)
            for k in range(KS):
                pltpu.make_async_copy(crows.at[k], acc.at[ekidx.at[k]],
                                      ssem).wait()
            return 0

        lax.fori_loop(0, (off + G - 1) >> 7, accum, 0)

        plsc.subcore_barrier()

        pltpu.async_copy(acc.at[pl.ds(s * SSZ, SSZ)],
                         out1.at[pl.ds(cb + s * SSZ, SSZ)], wsem)
        return off

    lax.fori_loop(0, CPS, chunk_body, jnp.int32(0))

    pltpu.make_async_copy(acc.at[pl.ds(s * SSZ, SSZ)],
                          out1.at[pl.ds(s * SSZ, SSZ)], wsem).wait()


def kernel(coords, vals):
    xcol = coords[:, 0]
    ycol = coords[:, 1]
    vals1 = vals.reshape(N // 128, 128, KS).transpose(0, 2, 1).reshape(VE)
    out1 = _scatter_kernel(xcol, ycol, vals1)
    return out1.reshape(H, W // 128, KS, 128).transpose(0, 1, 3, 2).reshape(
        H, W, KS)


# speedup vs baseline: 6.1768x; 1.2049x over previous
import functools

import jax
import jax.numpy as jnp
from jax import lax
from jax.experimental import pallas as pl
from jax.experimental.pallas import tpu as pltpu
from jax.experimental.pallas import tpu_sc as plsc

H, W, KS = 2048, 2048, 8
N = 65536
E = H * W * KS
VE = N * KS
NC, NS, L = 2, 16, 16
CHUNK = 1048576
NCHUNK = E // CHUNK
CPS = NCHUNK // NC
SSZ = CHUNK // NS
PTS = N // NS
ZROWS = 8192
G = 128
TRASH = CHUNK

_mesh = plsc.VectorSubcoreMesh(
    core_axis_name="c", subcore_axis_name="s", num_cores=NC, num_subcores=NS
)


def _scalar(v16, i):
    return lax.squeeze(lax.slice(v16, (i,), (i + 1,)), (0,))


@functools.partial(
    pl.kernel,
    out_type=jax.ShapeDtypeStruct((E,), jnp.float32),
    mesh=_mesh,
    scratch_types=[
        pltpu.VMEM((PTS,), jnp.int32),
        pltpu.VMEM((PTS,), jnp.int32),
        pltpu.VMEM((PTS,), jnp.int32),
        pltpu.VMEM((PTS,), jnp.int32),
        pltpu.VMEM((PTS,), jnp.int32),
        pltpu.VMEM((CPS * L,), jnp.int32),
        pltpu.VMEM((CPS + L,), jnp.int32),
        pltpu.VMEM((KS, G), jnp.int32),
        pltpu.VMEM((KS, G), jnp.int32),
        pltpu.VMEM((KS, G), jnp.float32),
        pltpu.VMEM((ZROWS,), jnp.float32),
        pltpu.VMEM_SHARED((CHUNK + NS * 1024,), jnp.float32),
        pltpu.SemaphoreType.DMA,
        pltpu.SemaphoreType.DMA,
        pltpu.SemaphoreType.DMA,
        pltpu.SemaphoreType.DMA,
    ],
    compiler_params=pltpu.CompilerParams(needs_layout_passes=False),
)
def _scatter_kernel(xcol, ycol, vals1, out1, xs_v, ys_v, eb_v, vb_v, bli,
                    roff, binst, ekidx, vkidx, crows, zbuf, acc,
                    gsem, ssem, zsem, wsem):
    c = lax.axis_index("c")
    s = lax.axis_index("s")
    iota = lax.iota(jnp.int32, L)
    zero_i = jnp.zeros((L,), jnp.int32)
    zero_f = jnp.zeros((L,), jnp.float32)

    pltpu.sync_copy(xcol.at[pl.ds(s * PTS, PTS)], xs_v)
    pltpu.sync_copy(ycol.at[pl.ds(s * PTS, PTS)], ys_v)

    def zh(i, _):
        plsc.store_scatter(roff, [iota + i * L], zero_i)
        return 0

    lax.fori_loop(0, CPS, zh, 0)

    cb0 = c * CPS * CHUNK

    def mk_base(i, _):
        lanes = iota + i * L
        xs = plsc.load_gather(xs_v, [lanes])
        ys = plsc.load_gather(ys_v, [lanes])
        eb = xs * (W * KS) + ((ys >> 7) << 10) + (ys & 127)
        plsc.store_scatter(eb_v, [lanes], eb)
        p = lanes + s * PTS
        plsc.store_scatter(vb_v, [lanes], ((p >> 7) << 10) + (p & 127))
        cid = (eb - cb0) >> 20
        m = (cid >= 0) & (cid < CPS)
        hidx = (cid << 4) + iota
        h = plsc.load_gather(roff, [hidx], mask=m)
        plsc.store_scatter(roff, [hidx], h + 1, mask=m)
        return 0

    lax.fori_loop(0, PTS // L, mk_base, 0)

    base = jnp.int32(0)
    for b in range(CPS):
        plsc.store_scatter(binst, [zero_i + b], zero_i + base,
                           mask=(iota == 0))
        row = plsc.load_gather(roff, [zero_i + (b << 4) + iota])
        cs = plsc.cumsum(row)
        plsc.store_scatter(roff, [zero_i + (b << 4) + iota],
                           base + cs - row)
        base = base + _scalar(cs, L - 1)
    plsc.store_scatter(binst, [zero_i + CPS], zero_i + base,
                       mask=(iota == 0))

    def binify(i, _):
        lanes = iota + i * L
        eb = plsc.load_gather(eb_v, [lanes])
        cid = (eb - cb0) >> 20
        m = (cid >= 0) & (cid < CPS)
        hidx = (cid << 4) + iota
        pos = plsc.load_gather(roff, [hidx], mask=m)
        plsc.store_scatter(roff, [hidx], pos + 1, mask=m)
        plsc.store_scatter(bli, [pos], lanes, mask=m)
        return 0

    lax.fori_loop(0, PTS // L, binify, 0)

    def mk_zero(i, _):
        plsc.store_scatter(zbuf, [iota + i * L], zero_f)
        return 0

    lax.fori_loop(0, ZROWS // L, mk_zero, 0)

    def z0(j, _):
        pltpu.sync_copy(zbuf, acc.at[pl.ds(s * SSZ + j * ZROWS, ZROWS)])
        return 0

    lax.fori_loop(0, SSZ // ZROWS, z0, 0)

    pltpu.async_copy(acc.at[pl.ds(s * SSZ, SSZ)],
                     out1.at[pl.ds(cb0 + s * SSZ, SSZ)], wsem)

    def expand(start, end, cb, g):
        for j in range(G // L):
            lanes = start + g * G + j * L + iota
            valid = lanes < end
            li = plsc.load_gather(bli, [lanes], mask=valid)
            e16 = plsc.load_gather(eb_v, [li], mask=valid) - cb
            v16 = plsc.load_gather(vb_v, [li], mask=valid)
            e16 = jnp.where(valid, e16, TRASH + s * 1024)
            v16 = jnp.where(valid, v16, s * (PTS * KS))
            for k in range(KS):
                plsc.store_scatter(ekidx, [zero_i + k, j * L + iota],
                                   e16 + k * G)
                plsc.store_scatter(vkidx, [zero_i + k, j * L + iota],
                                   v16 + k * G)

    def chunk_body(lc, _):
        cb = cb0 + lc * CHUNK
        st16 = plsc.load_gather(binst, [zero_i + lc])
        en16 = plsc.load_gather(binst, [zero_i + lc + 1])
        start, end = _scalar(st16, 0), _scalar(en16, 0)
        lcp = jnp.maximum(lc - 1, 0)
        pst16 = plsc.load_gather(binst, [zero_i + lcp])
        pen16 = plsc.load_gather(binst, [zero_i + lcp + 1])
        pstart, pend = _scalar(pst16, 0), _scalar(pen16, 0)
        pcnt = jnp.where(lc > 0, pend - pstart, 0)

        pltpu.make_async_copy(acc.at[pl.ds(s * SSZ, SSZ)],
                              out1.at[pl.ds(s * SSZ, SSZ)], wsem).wait()
        plsc.subcore_barrier()

        def zt(g, _):
            expand(pstart, pend, cb - CHUNK, g)
            for k in range(KS):
                pltpu.async_copy(zbuf.at[pl.ds(0, G)],
                                 acc.at[ekidx.at[k]], zsem)
            for k in range(KS):
                pltpu.make_async_copy(zbuf.at[pl.ds(0, G)],
                                      acc.at[ekidx.at[k]], zsem).wait()
            return 0

        lax.fori_loop(0, (pcnt + G - 1) >> 7, zt, 0)

        plsc.subcore_barrier()

        def accum(g, _):
            expand(start, end, cb, g)
            for k in range(KS):
                pltpu.async_copy(vals1.at[vkidx.at[k]], crows.at[k], gsem)
            for k in range(KS):
                pltpu.make_async_copy(vals1.at[vkidx.at[k]], crows.at[k],
                                      gsem).wait()
            for k in range(KS):
                pltpu.async_copy(crows.at[k], acc.at[ekidx.at[k]], ssem,
                                 add=True)
            for k in range(KS):
                pltpu.make_async_copy(crows.at[k], acc.at[ekidx.at[k]],
                                      ssem).wait()
            return 0

        lax.fori_loop(0, (end - start + G - 1) >> 7, accum, 0)

        plsc.subcore_barrier()

        pltpu.async_copy(acc.at[pl.ds(s * SSZ, SSZ)],
                         out1.at[pl.ds(cb + s * SSZ, SSZ)], wsem)
        return 0

    lax.fori_loop(0, CPS, chunk_body, 0)

    pltpu.make_async_copy(acc.at[pl.ds(s * SSZ, SSZ)],
                          out1.at[pl.ds(s * SSZ, SSZ)], wsem).wait()


def kernel(coords, vals):
    xcol = coords[:, 0]
    ycol = coords[:, 1]
    vals1 = vals.reshape(N // 128, 128, KS).transpose(0, 2, 1).reshape(VE)
    out1 = _scatter_kernel(xcol, ycol, vals1)
    return out1.reshape(H, W // 128, KS, 128).transpose(0, 1, 3, 2).reshape(
        H, W, KS)


# speedup vs baseline: 6.2587x; 1.0133x over previous
import functools

import jax
import jax.numpy as jnp
from jax import lax
from jax.experimental import pallas as pl
from jax.experimental.pallas import tpu as pltpu
from jax.experimental.pallas import tpu_sc as plsc

H, W, KS = 2048, 2048, 8
N = 65536
E = H * W * KS
VE = N * KS
NC, NS, L = 2, 16, 16
CHUNK = 1048576
NCHUNK = E // CHUNK
CPS = NCHUNK // NC
SSZ = CHUNK // NS
PTS = N // NS
ZROWS = 8192
G = 128
TRASH = CHUNK

_mesh = plsc.VectorSubcoreMesh(
    core_axis_name="c", subcore_axis_name="s", num_cores=NC, num_subcores=NS
)


def _scalar(v16, i):
    return lax.squeeze(lax.slice(v16, (i,), (i + 1,)), (0,))


@functools.partial(
    pl.kernel,
    out_type=jax.ShapeDtypeStruct((E,), jnp.float32),
    mesh=_mesh,
    scratch_types=[
        pltpu.VMEM((PTS,), jnp.int32),
        pltpu.VMEM((PTS,), jnp.int32),
        pltpu.VMEM((PTS,), jnp.int32),
        pltpu.VMEM((PTS,), jnp.int32),
        pltpu.VMEM((PTS,), jnp.int32),
        pltpu.VMEM((CPS * L,), jnp.int32),
        pltpu.VMEM((CPS + L,), jnp.int32),
        pltpu.VMEM((G,), jnp.int32),
        pltpu.VMEM((G,), jnp.int32),
        pltpu.VMEM((KS, G), jnp.float32),
        pltpu.VMEM((ZROWS,), jnp.float32),
        pltpu.VMEM_SHARED((CHUNK + NS * 1024,), jnp.float32),
        pltpu.SemaphoreType.DMA,
        pltpu.SemaphoreType.DMA,
        pltpu.SemaphoreType.DMA,
        pltpu.SemaphoreType.DMA,
    ],
    compiler_params=pltpu.CompilerParams(needs_layout_passes=False),
)
def _scatter_kernel(xcol, ycol, vals1, out1, xs_v, ys_v, eb_v, vb_v, bli,
                    roff, binst, eidx, vidx, crows, zbuf, acc,
                    gsem, ssem, zsem, wsem):
    c = lax.axis_index("c")
    s = lax.axis_index("s")
    iota = lax.iota(jnp.int32, L)
    zero_i = jnp.zeros((L,), jnp.int32)
    zero_f = jnp.zeros((L,), jnp.float32)

    pltpu.sync_copy(xcol.at[pl.ds(s * PTS, PTS)], xs_v)
    pltpu.sync_copy(ycol.at[pl.ds(s * PTS, PTS)], ys_v)

    def zh(i, _):
        plsc.store_scatter(roff, [iota + i * L], zero_i)
        return 0

    lax.fori_loop(0, CPS, zh, 0)

    cb0 = c * CPS * CHUNK

    def mk_base(i, _):
        lanes = iota + i * L
        xs = plsc.load_gather(xs_v, [lanes])
        ys = plsc.load_gather(ys_v, [lanes])
        eb = xs * (W * KS) + ((ys >> 7) << 10) + (ys & 127)
        plsc.store_scatter(eb_v, [lanes], eb)
        p = lanes + s * PTS
        plsc.store_scatter(vb_v, [lanes], ((p >> 7) << 10) + (p & 127))
        cid = (eb - cb0) >> 20
        m = (cid >= 0) & (cid < CPS)
        hidx = (cid << 4) + iota
        h = plsc.load_gather(roff, [hidx], mask=m)
        plsc.store_scatter(roff, [hidx], h + 1, mask=m)
        return 0

    lax.fori_loop(0, PTS // L, mk_base, 0)

    base = jnp.int32(0)
    for b in range(CPS):
        plsc.store_scatter(binst, [zero_i + b], zero_i + base,
                           mask=(iota == 0))
        row = plsc.load_gather(roff, [zero_i + (b << 4) + iota])
        cs = plsc.cumsum(row)
        plsc.store_scatter(roff, [zero_i + (b << 4) + iota],
                           base + cs - row)
        base = base + _scalar(cs, L - 1)
    plsc.store_scatter(binst, [zero_i + CPS], zero_i + base,
                       mask=(iota == 0))

    def binify(i, _):
        lanes = iota + i * L
        eb = plsc.load_gather(eb_v, [lanes])
        cid = (eb - cb0) >> 20
        m = (cid >= 0) & (cid < CPS)
        hidx = (cid << 4) + iota
        pos = plsc.load_gather(roff, [hidx], mask=m)
        plsc.store_scatter(roff, [hidx], pos + 1, mask=m)
        plsc.store_scatter(bli, [pos], lanes, mask=m)
        return 0

    lax.fori_loop(0, PTS // L, binify, 0)

    def mk_zero(i, _):
        plsc.store_scatter(zbuf, [iota + i * L], zero_f)
        return 0

    lax.fori_loop(0, ZROWS // L, mk_zero, 0)

    def z0(j, _):
        pltpu.sync_copy(zbuf, acc.at[pl.ds(s * SSZ + j * ZROWS, ZROWS)])
        return 0

    lax.fori_loop(0, SSZ // ZROWS, z0, 0)

    pltpu.async_copy(acc.at[pl.ds(s * SSZ, SSZ)],
                     out1.at[pl.ds(cb0 + s * SSZ, SSZ)], wsem)

    ACCL = CHUNK + NS * 1024
    KPAD = (KS - 1) * G

    def expand(start, end, cb, g):
        for j in range(G // L):
            lanes = start + g * G + j * L + iota
            valid = lanes < end
            li = plsc.load_gather(bli, [lanes], mask=valid)
            e16 = plsc.load_gather(eb_v, [li], mask=valid) - cb
            v16 = plsc.load_gather(vb_v, [li], mask=valid)
            e16 = jnp.where(valid, e16, TRASH + s * 1024)
            v16 = jnp.where(valid, v16, s * (PTS * KS))
            plsc.store_scatter(eidx, [j * L + iota], e16)
            plsc.store_scatter(vidx, [j * L + iota], v16)

    def chunk_body(lc, _):
        cb = cb0 + lc * CHUNK
        st16 = plsc.load_gather(binst, [zero_i + lc])
        en16 = plsc.load_gather(binst, [zero_i + lc + 1])
        start, end = _scalar(st16, 0), _scalar(en16, 0)
        lcp = jnp.maximum(lc - 1, 0)
        pst16 = plsc.load_gather(binst, [zero_i + lcp])
        pen16 = plsc.load_gather(binst, [zero_i + lcp + 1])
        pstart, pend = _scalar(pst16, 0), _scalar(pen16, 0)
        pcnt = jnp.where(lc > 0, pend - pstart, 0)

        pltpu.make_async_copy(acc.at[pl.ds(s * SSZ, SSZ)],
                              out1.at[pl.ds(s * SSZ, SSZ)], wsem).wait()
        plsc.subcore_barrier()

        def zt(g, _):
            expand(pstart, pend, cb - CHUNK, g)
            for k in range(KS):
                pltpu.async_copy(
                    zbuf.at[pl.ds(0, G)],
                    acc.at[pl.ds(k * G, ACCL - KPAD)].at[eidx], zsem)
            for k in range(KS):
                pltpu.make_async_copy(
                    zbuf.at[pl.ds(0, G)],
                    acc.at[pl.ds(k * G, ACCL - KPAD)].at[eidx], zsem).wait()
            return 0

        lax.fori_loop(0, (pcnt + G - 1) >> 7, zt, 0)

        plsc.subcore_barrier()

        def accum(g, _):
            expand(start, end, cb, g)
            for k in range(KS):
                pltpu.async_copy(
                    vals1.at[pl.ds(k * G, VE - KPAD)].at[vidx],
                    crows.at[k], gsem)
            for k in range(KS):
                pltpu.make_async_copy(
                    vals1.at[pl.ds(k * G, VE - KPAD)].at[vidx],
                    crows.at[k], gsem).wait()
            for k in range(KS):
                pltpu.async_copy(
                    crows.at[k],
                    acc.at[pl.ds(k * G, ACCL - KPAD)].at[eidx], ssem,
                    add=True)
            for k in range(KS):
                pltpu.make_async_copy(
                    crows.at[k],
                    acc.at[pl.ds(k * G, ACCL - KPAD)].at[eidx], ssem).wait()
            return 0

        lax.fori_loop(0, (end - start + G - 1) >> 7, accum, 0)

        plsc.subcore_barrier()

        pltpu.async_copy(acc.at[pl.ds(s * SSZ, SSZ)],
                         out1.at[pl.ds(cb + s * SSZ, SSZ)], wsem)
        return 0

    lax.fori_loop(0, CPS, chunk_body, 0)

    pltpu.make_async_copy(acc.at[pl.ds(s * SSZ, SSZ)],
                          out1.at[pl.ds(s * SSZ, SSZ)], wsem).wait()


def kernel(coords, vals):
    xcol = coords[:, 0]
    ycol = coords[:, 1]
    vals1 = vals.reshape(N // 128, 128, KS).transpose(0, 2, 1).reshape(VE)
    out1 = _scatter_kernel(xcol, ycol, vals1)
    return out1.reshape(H, W // 128, KS, 128).transpose(0, 1, 3, 2).reshape(
        H, W, KS)
